# baseline (speedup 1.0000x reference)
import functools

import jax
import jax.numpy as jnp
from jax import lax
from jax.experimental import pallas as pl
from jax.experimental.pallas import tpu as pltpu
from jax.experimental.pallas import tpu_sc as plsc

BATCH = 16384
EMBED_DIM = 64
NUM_CORES = 2
NUM_SUBCORES = 16
NUM_WORKERS = NUM_CORES * NUM_SUBCORES
BPW = BATCH // NUM_WORKERS
CHUNK = 128
LANES = 16


def _dot_body(user_hbm, mission_hbm, utab_hbm, mtab_hbm, out_hbm,
              uidx, midx, urows, mrows, tmat, out_v, sem):
    wid = lax.axis_index("s") * NUM_CORES + lax.axis_index("c")
    base = wid * BPW

    pltpu.sync_copy(user_hbm.at[pl.ds(base, BPW)], uidx)
    pltpu.sync_copy(mission_hbm.at[pl.ds(base, BPW)], midx)

    copies = []
    for c in range(BPW // CHUNK):
        sl = pl.ds(c * CHUNK, CHUNK)
        copies.append(pltpu.async_copy(utab_hbm.at[uidx.at[sl]], urows.at[sl], sem))
        copies.append(pltpu.async_copy(mtab_hbm.at[midx.at[sl]], mrows.at[sl], sem))
    for cp in copies:
        cp.wait()

    def blk_body(blk, carry):
        col = lax.iota(jnp.int32, LANES) * LANES
        for j in range(LANES):
            r = blk * LANES + j
            t = jnp.zeros((LANES,), jnp.float32)
            for k in range(EMBED_DIM // LANES):
                sl = pl.ds(k * LANES, LANES)
                t = t + urows[r, sl] * mrows[r, sl]
            plsc.store_scatter(tmat, [col + j], t)
        acc = jnp.zeros((LANES,), jnp.float32)
        for i in range(LANES):
            acc = acc + tmat[pl.ds(i * LANES, LANES)]
        out_v[pl.ds(blk * LANES, LANES)] = acc
        return carry

    lax.fori_loop(0, BPW // LANES, blk_body, 0)

    pltpu.sync_copy(out_v, out_hbm.at[pl.ds(base, BPW)])


@functools.partial(jax.jit, static_argnames=())
def kernel(user, mission, user_table, mission_table):
    mesh = plsc.VectorSubcoreMesh(core_axis_name="c", subcore_axis_name="s")
    run = functools.partial(
        pl.kernel,
        mesh=mesh,
        compiler_params=pltpu.CompilerParams(
            needs_layout_passes=False, use_tc_tiling_on_sc=False),
        out_type=jax.ShapeDtypeStruct((BATCH,), jnp.float32),
        scratch_types=[
            pltpu.VMEM((BPW,), jnp.int32),
            pltpu.VMEM((BPW,), jnp.int32),
            pltpu.VMEM((BPW, EMBED_DIM), jnp.float32),
            pltpu.VMEM((BPW, EMBED_DIM), jnp.float32),
            pltpu.VMEM((LANES * LANES,), jnp.float32),
            pltpu.VMEM((BPW,), jnp.float32),
            pltpu.SemaphoreType.DMA,
        ],
    )(_dot_body)
    return run(user, mission, user_table, mission_table)
